# baseline (device time: 652682 ns/iter reference)
import jax
import jax.numpy as jnp
from jax import lax
from jax.experimental import pallas as pl
from jax.experimental.pallas import tpu as pltpu

N_DEV = 4
M = 4096
K_SHARD = 1024
N_TOT = 8192
CHUNK = M // N_DEV
COL_BLK = 2048
HALF = COL_BLK // 2
HALFQ = HALF // 2
NBLK = N_TOT // COL_BLK
SLOT = [3, 0, 1, 2, 0, 1]


def _mod(e):
    return lax.rem(e, N_DEV)


def _body(x_ref, w_ref, s_ref, oi_ref, o_ref, sbuf, rbuf, pnext, wstage, wbf,
          send_sems, recv_sems, store_sems, wsem,
          credit_r, credit_l, cseed_r, cseed_l):
    d = lax.axis_index("i")
    left = _mod(d + N_DEV - 1)
    right = _mod(d + 1)

    scale = s_ref[0, 0]

    def sig(sem, dirn):
        nbr = left if dirn == 0 else right
        pl.semaphore_signal(sem, inc=1, device_id=(nbr,),
                            device_id_type=pl.DeviceIdType.MESH)

    def wload(b):
        return pltpu.make_async_copy(
            w_ref.at[:, pl.ds(b * COL_BLK, COL_BLK)], wstage, wsem)

    def pblock(b, dirn, c):
        return (jnp.dot(
            x_ref[pl.ds(c * CHUNK, CHUNK), :].astype(jnp.bfloat16),
            wbf[b % 2, :, dirn * HALF:dirn * HALF + HALF],
            preferred_element_type=jnp.float32,
        ) * scale).astype(jnp.bfloat16)

    def mk(b, dirn, k, q, src_ref=None):
        tgt = right if dirn == 0 else left
        slot = SLOT[k]
        qsl = pl.ds(q * HALFQ, HALFQ)
        return pltpu.make_async_remote_copy(
            src_ref=(sbuf.at[b % 2, dirn, :, qsl]
                     if src_ref is None else src_ref),
            dst_ref=rbuf.at[dirn, slot, :, qsl],
            send_sem=send_sems.at[dirn, slot, q],
            recv_sem=recv_sems.at[dirn, slot, q],
            device_id=(tgt,),
            device_id_type=pl.DeviceIdType.MESH,
        )

    def rs_recv(dirn, s):
        return _mod(d + 3 - s) if dirn == 0 else _mod(d + 1 + s)

    def ag_recv(dirn, t):
        return _mod(d + 4 - t) if dirn == 0 else _mod(d + t)

    bar = pltpu.get_barrier_semaphore()
    for nbr in (left, right):
        pl.semaphore_signal(bar, inc=1, device_id=(nbr,),
                            device_id_type=pl.DeviceIdType.MESH)
    pl.semaphore_wait(bar, 2)
    sig(credit_r, 0)
    sig(credit_l, 1)
    sig(cseed_r, 0)
    sig(cseed_l, 1)

    wl = wload(0)
    wl.start()
    wl.wait()
    wbf[0] = wstage[...].astype(jnp.bfloat16)
    if NBLK > 1:
        wload(1).start()

    for dirn in (0, 1):
        pl.semaphore_wait(cseed_r if dirn == 0 else cseed_l, 1)
        sbuf[0, dirn] = pblock(0, dirn, d)
        for q in (0, 1):
            mk(0, dirn, 0, q).start()
    for dirn in (0, 1):
        pnext[dirn] = pblock(0, dirn, rs_recv(dirn, 0))

    for b in range(NBLK):
        p = b % 2
        store_dmas = []

        def store(dirn, k, src_ref, c, b=b):
            dma = pltpu.make_async_copy(
                src_ref,
                o_ref.at[pl.ds(c * CHUNK, CHUNK),
                         pl.ds(b * COL_BLK + dirn * HALF, HALF)],
                store_sems.at[dirn, k],
            )
            dma.start()
            store_dmas.append(dma)

        pl.semaphore_wait(credit_r, 1)
        pl.semaphore_wait(credit_l, 1)

        for dirn in (0, 1):
            for q in (0, 1):
                lo, hi = q * HALFQ, (q + 1) * HALFQ
                mk(b, dirn, 0, q).wait()
                sbuf[p, dirn, :, lo:hi] = (
                    rbuf[dirn, 3, :, lo:hi] + pnext[dirn, :, lo:hi])
                mk(b, dirn, 1, q).start()
            if b < NBLK - 1:
                sig(cseed_r if dirn == 0 else cseed_l, dirn)
            pnext[dirn] = pblock(b, dirn, rs_recv(dirn, 1))

        for dirn in (0, 1):
            for q in (0, 1):
                lo, hi = q * HALFQ, (q + 1) * HALFQ
                mk(b, dirn, 1, q).wait()
                sbuf[p, dirn, :, lo:hi] = (
                    rbuf[dirn, 0, :, lo:hi] + pnext[dirn, :, lo:hi])
                mk(b, dirn, 2, q).start()
            pnext[dirn] = pblock(b, dirn, rs_recv(dirn, 2))

        if b < NBLK - 1:
            wload(b + 1).wait()
            wbf[(b + 1) % 2] = wstage[...].astype(jnp.bfloat16)
            if b + 2 < NBLK:
                wload(b + 2).start()

        for dirn in (0, 1):
            for q in (0, 1):
                lo, hi = q * HALFQ, (q + 1) * HALFQ
                mk(b, dirn, 2, q).wait()
                sbuf[p, dirn, :, lo:hi] = (
                    rbuf[dirn, 1, :, lo:hi] + pnext[dirn, :, lo:hi])
                mk(b, dirn, 3, q).start()
            store(dirn, 0, sbuf.at[p, dirn], rs_recv(dirn, 2))

        if b < NBLK - 1:
            for dirn in (0, 1):
                pl.semaphore_wait(cseed_r if dirn == 0 else cseed_l, 1)
                sbuf[1 - p, dirn] = pblock(b + 1, dirn, d)
                for q in (0, 1):
                    mk(b + 1, dirn, 0, q).start()
            for dirn in (0, 1):
                pnext[dirn] = pblock(b + 1, dirn, rs_recv(dirn, 0))

        for t in (0, 1):
            for dirn in (0, 1):
                fslot = SLOT[3 + t]
                for q in (0, 1):
                    qsl = pl.ds(q * HALFQ, HALFQ)
                    mk(b, dirn, 3 + t, q).wait()
                    mk(b, dirn, 4 + t,
                       q, src_ref=rbuf.at[dirn, fslot, :, qsl]).start()
                store(dirn, 1 + t, rbuf.at[dirn, fslot], ag_recv(dirn, t))
        for dirn in (0, 1):
            for q in (0, 1):
                mk(b, dirn, 5, q).wait()
            store(dirn, 3, rbuf.at[dirn, SLOT[5]], ag_recv(dirn, 2))

        for dma in store_dmas:
            dma.wait()
        if b < NBLK - 1:
            sig(credit_r, 0)
            sig(credit_l, 1)


def _fused(xb, wb, scale):
    out_init = jnp.zeros((M, N_TOT), jnp.bfloat16)
    return pl.pallas_call(
        _body,
        in_specs=[
            pl.BlockSpec(memory_space=pltpu.VMEM),
            pl.BlockSpec(memory_space=pltpu.MemorySpace.HBM),
            pl.BlockSpec(memory_space=pltpu.VMEM),
            pl.BlockSpec(memory_space=pltpu.MemorySpace.HBM),
        ],
        input_output_aliases={3: 0},
        out_specs=pl.BlockSpec(memory_space=pltpu.MemorySpace.HBM),
        out_shape=jax.ShapeDtypeStruct((M, N_TOT), jnp.bfloat16),
        scratch_shapes=[
            pltpu.VMEM((2, 2, CHUNK, HALF), jnp.bfloat16),
            pltpu.VMEM((2, 4, CHUNK, HALF), jnp.bfloat16),
            pltpu.VMEM((2, CHUNK, HALF), jnp.bfloat16),
            pltpu.VMEM((K_SHARD, COL_BLK), jnp.float32),
            pltpu.VMEM((2, K_SHARD, COL_BLK), jnp.bfloat16),
            pltpu.SemaphoreType.DMA((2, 4, 2)),
            pltpu.SemaphoreType.DMA((2, 4, 2)),
            pltpu.SemaphoreType.DMA((2, 4)),
            pltpu.SemaphoreType.DMA,
            pltpu.SemaphoreType.REGULAR,
            pltpu.SemaphoreType.REGULAR,
            pltpu.SemaphoreType.REGULAR,
            pltpu.SemaphoreType.REGULAR,
        ],
        compiler_params=pltpu.CompilerParams(
            collective_id=0,
            vmem_limit_bytes=67_000_000,
        ),
    )(xb, wb, scale, out_init)


def kernel(x, w_mat, scale_x, scale_w):
    xb = x.astype(jnp.float32)
    wb = w_mat.astype(jnp.float32)
    scale = (scale_x.astype(jnp.float32)
             * scale_w.astype(jnp.float32)).reshape(1, 1)
    return _fused(xb, wb, scale)


# device time: 630263 ns/iter; 1.0356x vs baseline; 1.0356x over previous
import jax
import jax.numpy as jnp
from jax import lax
from jax.experimental import pallas as pl
from jax.experimental.pallas import tpu as pltpu

N_DEV = 4
M = 4096
K_SHARD = 1024
N_TOT = 8192
CHUNK = M // N_DEV
COL_BLK = 2048
HALF = COL_BLK // 2
HALFQ = HALF // 2
NBLK = N_TOT // COL_BLK
SLOT = [3, 0, 1, 2, 0, 1]


def _mod(e):
    return lax.rem(e, N_DEV)


def _body(x_ref, w_ref, s_ref, o_ref, sbuf, rbuf, pnext, wstage, wbf,
          send_sems, recv_sems, store_sems, wsem,
          credit_r, credit_l, cseed_r, cseed_l):
    d = lax.axis_index("i")
    left = _mod(d + N_DEV - 1)
    right = _mod(d + 1)

    scale = s_ref[0, 0]

    def sig(sem, dirn):
        nbr = left if dirn == 0 else right
        pl.semaphore_signal(sem, inc=1, device_id=(nbr,),
                            device_id_type=pl.DeviceIdType.MESH)

    def wload(b):
        return pltpu.make_async_copy(
            w_ref.at[:, pl.ds(b * COL_BLK, COL_BLK)], wstage, wsem)

    def pblock(b, dirn, c):
        return (jnp.dot(
            x_ref[pl.ds(c * CHUNK, CHUNK), :].astype(jnp.bfloat16),
            wbf[b % 2, :, dirn * HALF:dirn * HALF + HALF],
            preferred_element_type=jnp.float32,
        ) * scale).astype(jnp.bfloat16)

    def mk(b, dirn, k, q, src_ref=None):
        tgt = right if dirn == 0 else left
        slot = SLOT[k]
        qsl = pl.ds(q * HALFQ, HALFQ)
        return pltpu.make_async_remote_copy(
            src_ref=(sbuf.at[b % 2, dirn, :, qsl]
                     if src_ref is None else src_ref),
            dst_ref=rbuf.at[dirn, slot, :, qsl],
            send_sem=send_sems.at[dirn, slot, q],
            recv_sem=recv_sems.at[dirn, slot, q],
            device_id=(tgt,),
            device_id_type=pl.DeviceIdType.MESH,
        )

    def rs_recv(dirn, s):
        return _mod(d + 3 - s) if dirn == 0 else _mod(d + 1 + s)

    def ag_recv(dirn, t):
        return _mod(d + 4 - t) if dirn == 0 else _mod(d + t)

    bar = pltpu.get_barrier_semaphore()
    for nbr in (left, right):
        pl.semaphore_signal(bar, inc=1, device_id=(nbr,),
                            device_id_type=pl.DeviceIdType.MESH)
    pl.semaphore_wait(bar, 2)
    sig(credit_r, 0)
    sig(credit_l, 1)
    sig(cseed_r, 0)
    sig(cseed_l, 1)

    wl = wload(0)
    wl.start()
    wl.wait()
    wbf[0] = wstage[...].astype(jnp.bfloat16)
    if NBLK > 1:
        wload(1).start()

    for dirn in (0, 1):
        pl.semaphore_wait(cseed_r if dirn == 0 else cseed_l, 1)
        sbuf[0, dirn] = pblock(0, dirn, d)
        for q in (0, 1):
            mk(0, dirn, 0, q).start()
    for dirn in (0, 1):
        pnext[dirn] = pblock(0, dirn, rs_recv(dirn, 0))

    for b in range(NBLK):
        p = b % 2
        store_dmas = []

        def store(dirn, k, src_ref, c, b=b):
            dma = pltpu.make_async_copy(
                src_ref,
                o_ref.at[pl.ds(c * CHUNK, CHUNK),
                         pl.ds(b * COL_BLK + dirn * HALF, HALF)],
                store_sems.at[dirn, k],
            )
            dma.start()
            store_dmas.append(dma)

        pl.semaphore_wait(credit_r, 1)
        pl.semaphore_wait(credit_l, 1)

        for dirn in (0, 1):
            for q in (0, 1):
                lo, hi = q * HALFQ, (q + 1) * HALFQ
                mk(b, dirn, 0, q).wait()
                sbuf[p, dirn, :, lo:hi] = (
                    rbuf[dirn, 3, :, lo:hi] + pnext[dirn, :, lo:hi])
                mk(b, dirn, 1, q).start()
            if b < NBLK - 1:
                sig(cseed_r if dirn == 0 else cseed_l, dirn)
            pnext[dirn] = pblock(b, dirn, rs_recv(dirn, 1))

        for dirn in (0, 1):
            for q in (0, 1):
                lo, hi = q * HALFQ, (q + 1) * HALFQ
                mk(b, dirn, 1, q).wait()
                sbuf[p, dirn, :, lo:hi] = (
                    rbuf[dirn, 0, :, lo:hi] + pnext[dirn, :, lo:hi])
                mk(b, dirn, 2, q).start()
            pnext[dirn] = pblock(b, dirn, rs_recv(dirn, 2))

        if b < NBLK - 1:
            wload(b + 1).wait()
            wbf[(b + 1) % 2] = wstage[...].astype(jnp.bfloat16)
            if b + 2 < NBLK:
                wload(b + 2).start()

        for dirn in (0, 1):
            for q in (0, 1):
                lo, hi = q * HALFQ, (q + 1) * HALFQ
                mk(b, dirn, 2, q).wait()
                sbuf[p, dirn, :, lo:hi] = (
                    rbuf[dirn, 1, :, lo:hi] + pnext[dirn, :, lo:hi])
                mk(b, dirn, 3, q).start()
            store(dirn, 0, sbuf.at[p, dirn], rs_recv(dirn, 2))

        if b < NBLK - 1:
            for dirn in (0, 1):
                pl.semaphore_wait(cseed_r if dirn == 0 else cseed_l, 1)
                sbuf[1 - p, dirn] = pblock(b + 1, dirn, d)
                for q in (0, 1):
                    mk(b + 1, dirn, 0, q).start()
            for dirn in (0, 1):
                pnext[dirn] = pblock(b + 1, dirn, rs_recv(dirn, 0))

        for t in (0, 1):
            for dirn in (0, 1):
                fslot = SLOT[3 + t]
                for q in (0, 1):
                    qsl = pl.ds(q * HALFQ, HALFQ)
                    mk(b, dirn, 3 + t, q).wait()
                    mk(b, dirn, 4 + t,
                       q, src_ref=rbuf.at[dirn, fslot, :, qsl]).start()
                store(dirn, 1 + t, rbuf.at[dirn, fslot], ag_recv(dirn, t))
        for dirn in (0, 1):
            for q in (0, 1):
                mk(b, dirn, 5, q).wait()
            store(dirn, 3, rbuf.at[dirn, SLOT[5]], ag_recv(dirn, 2))

        for dma in store_dmas:
            dma.wait()
        if b < NBLK - 1:
            sig(credit_r, 0)
            sig(credit_l, 1)


def _fused(xb, wb, scale):
    return pl.pallas_call(
        _body,
        in_specs=[
            pl.BlockSpec(memory_space=pltpu.VMEM),
            pl.BlockSpec(memory_space=pltpu.MemorySpace.HBM),
            pl.BlockSpec(memory_space=pltpu.VMEM),
        ],
        out_specs=pl.BlockSpec(memory_space=pltpu.MemorySpace.HBM),
        out_shape=jax.ShapeDtypeStruct((M, N_TOT), jnp.bfloat16),
        scratch_shapes=[
            pltpu.VMEM((2, 2, CHUNK, HALF), jnp.bfloat16),
            pltpu.VMEM((2, 4, CHUNK, HALF), jnp.bfloat16),
            pltpu.VMEM((2, CHUNK, HALF), jnp.bfloat16),
            pltpu.VMEM((K_SHARD, COL_BLK), jnp.float32),
            pltpu.VMEM((2, K_SHARD, COL_BLK), jnp.bfloat16),
            pltpu.SemaphoreType.DMA((2, 4, 2)),
            pltpu.SemaphoreType.DMA((2, 4, 2)),
            pltpu.SemaphoreType.DMA((2, 4)),
            pltpu.SemaphoreType.DMA,
            pltpu.SemaphoreType.REGULAR,
            pltpu.SemaphoreType.REGULAR,
            pltpu.SemaphoreType.REGULAR,
            pltpu.SemaphoreType.REGULAR,
        ],
        compiler_params=pltpu.CompilerParams(
            collective_id=0,
            vmem_limit_bytes=67_000_000,
        ),
    )(xb, wb, scale)


def kernel(x, w_mat, scale_x, scale_w):
    xb = x.astype(jnp.float32)
    wb = w_mat.astype(jnp.float32)
    scale = (scale_x.astype(jnp.float32)
             * scale_w.astype(jnp.float32)).reshape(1, 1)
    return _fused(xb, wb, scale)
